# Initial kernel scaffold; baseline (speedup 1.0000x reference)
import jax
import jax.numpy as jnp
from jax.experimental import pallas as pl


def kernel(src_h, src_tw, src_tb, edge_h, edge_date, Wq, Wk, Wv, Wa, src_ln_g, src_ln_b, dst_ln_g, dst_ln_b, h_bias, skip, edge_index, edge_etype, dst_ntype):
    raise NotImplementedError("write your pallas kernel here")


# speedup vs baseline: 8.5651x; 8.5651x over previous
import functools
import math

import jax
import jax.numpy as jnp
from jax import lax
from jax.experimental import pallas as pl
from jax.experimental.pallas import tpu as pltpu
from jax.experimental.pallas import tpu_sc as plsc

N = 10000
E = 160000
IN_DIM = 128
OUT_DIM = 128
E_DIM = 16
TIME_DIM = 32
NUM_HEADS = 8
HEAD = OUT_DIM // NUM_HEADS
NUM_RELS = 8
NUM_NTYPES = 4
ROW = IN_DIM + 2 * TIME_DIM
CDIM = 144

NC = 2
NS = 16
NW = NC * NS
EPW = E // NW
GCHUNK = 200
NCHUNKS = EPW // GCHUNK
NPT = N // NS

@functools.cache
def _sc_mesh():
    return plsc.VectorSubcoreMesh(core_axis_name="c", subcore_axis_name="s",
                                  num_cores=NC, num_subcores=NS)


def _gather_body(table_hbm, srcidx_hbm, dstidx_hbm, out_s_hbm, out_d_hbm,
                 idx_v, rows_v, sem):
    cid = lax.axis_index("c")
    sid = lax.axis_index("s")
    base = (cid * NS + sid) * EPW

    def step(j, carry):
        off = base + j * GCHUNK
        pltpu.sync_copy(srcidx_hbm.at[pl.ds(off, GCHUNK)], idx_v)
        pltpu.async_copy(table_hbm.at[idx_v], rows_v, sem).wait()
        pltpu.sync_copy(rows_v, out_s_hbm.at[pl.ds(off, GCHUNK)])
        pltpu.sync_copy(dstidx_hbm.at[pl.ds(off, GCHUNK)], idx_v)
        pltpu.async_copy(table_hbm.at[idx_v], rows_v, sem).wait()
        pltpu.sync_copy(rows_v, out_d_hbm.at[pl.ds(off, GCHUNK)])
        return carry

    lax.fori_loop(0, NCHUNKS, step, 0)


@jax.jit
def _gather(table, src, dst):
    k = pl.kernel(
        _gather_body,
        out_type=(jax.ShapeDtypeStruct((E, ROW), jnp.float32),
                  jax.ShapeDtypeStruct((E, ROW), jnp.float32)),
        mesh=_sc_mesh(),
        scratch_types=[
            pltpu.VMEM((GCHUNK,), jnp.int32),
            pltpu.VMEM((GCHUNK, ROW), jnp.float32),
            pltpu.SemaphoreType.DMA,
        ],
        compiler_params=pltpu.CompilerParams(use_tc_tiling_on_sc=False),
    )
    return k(table, src, dst)


def _scatter_body(contrib_hbm, dstidx_hbm, zeros_hbm, out_hbm,
                  idx_v, rows_v, acc_sh):
    cid = lax.axis_index("c")
    sid = lax.axis_index("s")
    pltpu.sync_copy(zeros_hbm.at[pl.ds(sid * NPT, NPT)],
                    acc_sh.at[pl.ds(sid * NPT, NPT)])
    plsc.subcore_barrier()
    base = (cid * NS + sid) * EPW

    def step(j, carry):
        off = base + j * GCHUNK
        pltpu.sync_copy(dstidx_hbm.at[pl.ds(off, GCHUNK)], idx_v)
        pltpu.sync_copy(contrib_hbm.at[pl.ds(off, GCHUNK)], rows_v)
        pltpu.sync_copy(rows_v, acc_sh.at[idx_v], add=True)
        return carry

    lax.fori_loop(0, NCHUNKS, step, 0)
    plsc.subcore_barrier()
    pltpu.sync_copy(acc_sh.at[pl.ds(sid * NPT, NPT)],
                    out_hbm.at[cid, pl.ds(sid * NPT, NPT)])


@jax.jit
def _scatter(contrib, dst, zeros):
    k = pl.kernel(
        _scatter_body,
        out_type=jax.ShapeDtypeStruct((NC, N, CDIM), jnp.float32),
        mesh=_sc_mesh(),
        scratch_types=[
            pltpu.VMEM((GCHUNK,), jnp.int32),
            pltpu.VMEM((GCHUNK, CDIM), jnp.float32),
            pltpu.VMEM_SHARED((N, CDIM), jnp.float32),
        ],
        compiler_params=pltpu.CompilerParams(use_tc_tiling_on_sc=False),
    )
    return k(contrib, dst, zeros)


EBLK = 2000


def _layer_norm(x, g, b, eps=1e-5):
    mu = jnp.mean(x, axis=-1, keepdims=True)
    var = jnp.mean((x - mu) * (x - mu), axis=-1, keepdims=True)
    return (x - mu) * jax.lax.rsqrt(var + eps) * g + b


def _typed(x, et, w_ref, nrel):
    acc = jnp.zeros((x.shape[0], OUT_DIM), jnp.float32)
    for r in range(nrel):
        m = (et == r).astype(jnp.float32)
        acc = acc + m * jnp.dot(x, w_ref[r], preferred_element_type=jnp.float32)
    return acc


def _head_sum_mat():
    i = lax.broadcasted_iota(jnp.int32, (OUT_DIM, NUM_HEADS), 0)
    h = lax.broadcasted_iota(jnp.int32, (OUT_DIM, NUM_HEADS), 1)
    return (i // HEAD == h).astype(jnp.float32)


def _head_bcast_mat():
    h = lax.broadcasted_iota(jnp.int32, (NUM_HEADS, OUT_DIM), 0)
    i = lax.broadcasted_iota(jnp.int32, (NUM_HEADS, OUT_DIM), 1)
    return (i // HEAD == h).astype(jnp.float32)


def _edge_kernel_body(rs_ref, rd_ref, eh_ref, t_ref, et_ref,
                      wq_ref, wk_ref, wv_ref, gs_ref, bs_ref, gd_ref, bd_ref,
                      out_ref):
    rs = rs_ref[...]
    rd = rd_ref[...]
    t = t_ref[...]
    et = et_ref[...]
    hs = rs[:, :IN_DIM]
    dia_s = jnp.sin(rs[:, IN_DIM:IN_DIM + TIME_DIM] * t
                    + rs[:, IN_DIM + TIME_DIM:]) * hs[:, :TIME_DIM]
    xs = jnp.concatenate([dia_s, hs[:, TIME_DIM:], eh_ref[...]], axis=1)
    xs = _layer_norm(xs, gs_ref[...], bs_ref[...])
    hd = rd[:, :IN_DIM]
    dia_d = jnp.sin(rd[:, IN_DIM:IN_DIM + TIME_DIM] * t
                    + rd[:, IN_DIM + TIME_DIM:]) * hd[:, :TIME_DIM]
    xd = jnp.concatenate([dia_d, hd[:, TIME_DIM:]], axis=1)
    xd = _layer_norm(xd, gd_ref[...], bd_ref[...])

    q = _typed(xd, et, wq_ref, NUM_RELS)
    k = _typed(xs, et, wk_ref, NUM_RELS)
    v = _typed(xs, et, wv_ref, NUM_RELS)

    attn = jnp.dot(q * k, _head_sum_mat(),
                   preferred_element_type=jnp.float32) * (1.0 / math.sqrt(OUT_DIM))
    ex = jnp.exp(attn)
    vw = v * jnp.dot(ex, _head_bcast_mat(),
                     preferred_element_type=jnp.float32)
    out_ref[...] = jnp.concatenate(
        [ex, jnp.zeros((ex.shape[0], CDIM - OUT_DIM - NUM_HEADS), jnp.float32), vw],
        axis=1)


@jax.jit
def _edge_compute(rows_s, rows_d, edge_h, t_col, et_col, Wq, Wk, Wv,
                  gs, bs, gd, bd):
    grid = (E // EBLK,)
    return pl.pallas_call(
        _edge_kernel_body,
        grid=grid,
        in_specs=[
            pl.BlockSpec((EBLK, ROW), lambda i: (i, 0)),
            pl.BlockSpec((EBLK, ROW), lambda i: (i, 0)),
            pl.BlockSpec((EBLK, E_DIM), lambda i: (i, 0)),
            pl.BlockSpec((EBLK, 1), lambda i: (i, 0)),
            pl.BlockSpec((EBLK, 1), lambda i: (i, 0)),
            pl.BlockSpec((NUM_RELS, IN_DIM, OUT_DIM), lambda i: (0, 0, 0)),
            pl.BlockSpec((NUM_RELS, IN_DIM + E_DIM, OUT_DIM), lambda i: (0, 0, 0)),
            pl.BlockSpec((NUM_RELS, IN_DIM + E_DIM, OUT_DIM), lambda i: (0, 0, 0)),
            pl.BlockSpec((1, IN_DIM + E_DIM), lambda i: (0, 0)),
            pl.BlockSpec((1, IN_DIM + E_DIM), lambda i: (0, 0)),
            pl.BlockSpec((1, IN_DIM), lambda i: (0, 0)),
            pl.BlockSpec((1, IN_DIM), lambda i: (0, 0)),
        ],
        out_specs=pl.BlockSpec((EBLK, CDIM), lambda i: (i, 0)),
        out_shape=jax.ShapeDtypeStruct((E, CDIM), jnp.float32),
    )(rows_s, rows_d, edge_h, t_col, et_col, Wq, Wk, Wv, gs, bs, gd, bd)


NBLK = 1000


def _epi_kernel_body(p_ref, srch_ref, nt_ref, hb_ref, wa_ref, skip_ref, out_ref):
    s = p_ref[0] + p_ref[1]
    den = s[:, :NUM_HEADS]
    num = s[:, NUM_HEADS + (CDIM - OUT_DIM - NUM_HEADS):]
    den = jnp.where(den > 0.0, den, 1.0)
    h = num / jnp.dot(den, _head_bcast_mat(), preferred_element_type=jnp.float32)
    nt = nt_ref[...]
    onehot = (nt == lax.broadcasted_iota(jnp.int32, (1, NUM_NTYPES), 1)
              ).astype(jnp.float32)
    h = h + jnp.dot(onehot, hb_ref[...], preferred_element_type=jnp.float32)
    h2 = _typed(h, nt, wa_ref, NUM_NTYPES)
    al = jax.nn.sigmoid(jnp.dot(onehot, skip_ref[...],
                                preferred_element_type=jnp.float32))
    out_ref[...] = h2 * al + srch_ref[...] * (1.0 - al)


@jax.jit
def _epilogue(partials, src_h, nt_col, h_bias, Wa, skip_col):
    grid = (N // NBLK,)
    return pl.pallas_call(
        _epi_kernel_body,
        grid=grid,
        in_specs=[
            pl.BlockSpec((NC, NBLK, CDIM), lambda i: (0, i, 0)),
            pl.BlockSpec((NBLK, IN_DIM), lambda i: (i, 0)),
            pl.BlockSpec((NBLK, 1), lambda i: (i, 0)),
            pl.BlockSpec((NUM_NTYPES, OUT_DIM), lambda i: (0, 0)),
            pl.BlockSpec((NUM_NTYPES, OUT_DIM, OUT_DIM), lambda i: (0, 0, 0)),
            pl.BlockSpec((NUM_NTYPES, 1), lambda i: (0, 0)),
        ],
        out_specs=pl.BlockSpec((NBLK, OUT_DIM), lambda i: (i, 0)),
        out_shape=jax.ShapeDtypeStruct((N, OUT_DIM), jnp.float32),
    )(partials, src_h, nt_col, h_bias, Wa, skip_col)


def kernel(src_h, src_tw, src_tb, edge_h, edge_date, Wq, Wk, Wv, Wa,
           src_ln_g, src_ln_b, dst_ln_g, dst_ln_b, h_bias, skip,
           edge_index, edge_etype, dst_ntype):
    src = edge_index[0]
    dst = edge_index[1]
    table = jnp.concatenate([src_h, src_tw, src_tb], axis=1)
    rows_s, rows_d = _gather(table, src, dst)
    contrib = _edge_compute(
        rows_s, rows_d, edge_h,
        edge_date.reshape(E, 1), edge_etype.reshape(E, 1),
        Wq, Wk, Wv,
        src_ln_g.reshape(1, -1), src_ln_b.reshape(1, -1),
        dst_ln_g.reshape(1, -1), dst_ln_b.reshape(1, -1))
    partials = _scatter(contrib, dst, jnp.zeros((N, CDIM), jnp.float32))
    return _epilogue(partials, src_h, dst_ntype.reshape(N, 1),
                     h_bias, Wa, skip.reshape(NUM_NTYPES, 1))
